# baseline (speedup 1.0000x reference)
import jax
import jax.numpy as jnp
from jax import lax
from jax.experimental import pallas as pl
from jax.experimental.pallas import tpu as pltpu
from jax.experimental.pallas import tpu_sc as plsc

N_NODES = 10000
N_PAD = 10240
D = 128
DH = 64
E = 320000
NC, NS, L = 2, 16, 16
EPT = E // NS
K = 40
NCHUNK = EPT // K
NBUF = 5
NGRP = NCHUNK // NBUF
RPT = N_PAD // NS
CH = 80
NRCH = RPT // CH


def _rsqrt16(d):
    i = lax.bitcast_convert_type(d, jnp.int32)
    i = jnp.int32(0x5F3759DF) - lax.shift_right_logical(
        i, jnp.full((16,), 1, jnp.int32)
    )
    y = lax.bitcast_convert_type(i, jnp.float32)
    for _ in range(3):
        y = y * (1.5 - 0.5 * d * y * y)
    return jnp.where(d > 0.0, y, jnp.zeros((16,), jnp.float32))


def _bcast(scale_ref, i):
    return plsc.load_gather(scale_ref, [jnp.full((L,), i, jnp.int32)])


def _zero_buf(buf, rows):
    def body(r, carry):
        for f in range(DH // L):
            buf[r, pl.ds(f * L, L)] = jnp.zeros((L,), jnp.float32)
        return carry
    lax.fori_loop(0, rows, body, 0)


def _body(x_hbm, row_h, col_h, out_hbm,
          y_h, acc_s, deg_s,
          row_t, col_t, sbuf,
          ones_t, deg_t, dinv_t, dinv2_t, zvec,
          *bufsems):
    c = lax.axis_index("c")
    s = lax.axis_index("s")
    r0 = s * RPT
    f0 = c * DH
    bufs = bufsems[:NBUF]
    gsems = bufsems[NBUF:2 * NBUF]
    ssems = bufsems[2 * NBUF:3 * NBUF]

    pltpu.sync_copy(row_h.at[s], row_t)
    pltpu.sync_copy(col_h.at[s], col_t)

    for i in range(K // L):
        ones_t[pl.ds(i * L, L)] = jnp.ones((L,), jnp.float32)

    def zv(i, carry):
        zvec[pl.ds(i * L, L)] = jnp.zeros((L,), jnp.float32)
        return carry
    lax.fori_loop(0, RPT // L, zv, 0)

    pltpu.sync_copy(zvec, deg_s.at[pl.ds(r0, RPT)])
    plsc.subcore_barrier()

    def deg_fire(j, carry):
        pltpu.async_copy(ones_t, deg_s.at[col_t.at[j]], gsems[0], add=True)
        return carry
    lax.fori_loop(0, NCHUNK, deg_fire, 0)

    def deg_drain(j, carry):
        pltpu.make_async_copy(ones_t, deg_s.at[col_t.at[0]], gsems[0]).wait()
        return carry
    lax.fori_loop(0, NCHUNK, deg_drain, 0)
    plsc.subcore_barrier()

    pltpu.sync_copy(deg_s.at[pl.ds(r0, RPT)], deg_t)
    for i in range(RPT // L):
        sl = pl.ds(i * L, L)
        r = _rsqrt16(deg_t[sl])
        dinv_t[sl] = r
        dinv2_t[sl] = r * r

    def init_chunk(ch, carry):
        r = r0 + ch * CH

        @pl.when(r < N_NODES)
        def _():
            pltpu.sync_copy(x_hbm.at[pl.ds(r, CH), pl.ds(f0, DH)], sbuf)

            def sel(rr, carry2):
                v = _bcast(dinv_t, ch * CH + rr)
                for f in range(DH // L):
                    sl = pl.ds(f * L, L)
                    sbuf[rr, sl] = sbuf[rr, sl] * v
                return carry2
            lax.fori_loop(0, CH, sel, 0)
            pltpu.sync_copy(sbuf, y_h.at[c, pl.ds(r, CH), :])

        _zero_buf(sbuf, CH)

        @pl.when(r >= N_NODES)
        def _():
            pltpu.sync_copy(sbuf, y_h.at[c, pl.ds(r, CH), :])

        pltpu.sync_copy(sbuf, acc_s.at[pl.ds(r, CH), :])
        return carry
    lax.fori_loop(0, NRCH, init_chunk, 0)
    plsc.subcore_barrier()

    def gfire(j, b):
        pltpu.async_copy(y_h.at[c].at[row_t.at[j]], bufs[b], gsems[b])

    def gwait(b):
        pltpu.make_async_copy(y_h.at[c].at[row_t.at[0]], bufs[b], gsems[b]).wait()

    def sfire(j, b):
        pltpu.async_copy(bufs[b], acc_s.at[col_t.at[j]], ssems[b], add=True)

    def swait(b):
        pltpu.make_async_copy(bufs[b], acc_s.at[col_t.at[0]], ssems[b]).wait()

    def edge_pass():
        for b in range(NBUF):
            gfire(jnp.int32(b), b)

        def grp(g, carry):
            j0 = g * NBUF
            for b in range(NBUF):
                gwait(b)
                sfire(j0 + b, b)
            for b in range(NBUF):
                swait(b)

                @pl.when(j0 + b + NBUF < NCHUNK)
                def _():
                    gfire(j0 + b + NBUF, b)
            return carry
        lax.fori_loop(0, NGRP, grp, 0)

    for t in range(3):
        edge_pass()
        plsc.subcore_barrier()

        last = t == 2

        def scale_chunk(ch, carry):
            r = r0 + ch * CH

            @pl.when(r < N_NODES)
            def _():
                pltpu.sync_copy(acc_s.at[pl.ds(r, CH), :], sbuf)

                def rows(rr, carry2):
                    v = _bcast(dinv_t if last else dinv2_t, ch * CH + rr)
                    for f in range(DH // L):
                        sl = pl.ds(f * L, L)
                        sbuf[rr, sl] = sbuf[rr, sl] * v
                    return carry2
                lax.fori_loop(0, CH, rows, 0)
                if last:
                    pltpu.sync_copy(sbuf, out_hbm.at[pl.ds(r, CH), pl.ds(f0, DH)])
                else:
                    pltpu.sync_copy(sbuf, y_h.at[c, pl.ds(r, CH), :])

            if not last:
                _zero_buf(sbuf, CH)
                pltpu.sync_copy(sbuf, acc_s.at[pl.ds(r, CH), :])
            return carry
        lax.fori_loop(0, NRCH, scale_chunk, 0)
        if not last:
            plsc.subcore_barrier()


@jax.jit
def kernel(x, edge_index):
    ei = edge_index.astype(jnp.int32)
    row_h = ei[0].reshape(NS, NCHUNK, K)
    col_h = ei[1].reshape(NS, NCHUNK, K)

    kern = pl.kernel(
        _body,
        out_type=jax.ShapeDtypeStruct((N_NODES, D), jnp.float32),
        mesh=plsc.VectorSubcoreMesh(core_axis_name="c", subcore_axis_name="s"),
        compiler_params=pltpu.CompilerParams(
            needs_layout_passes=False, use_tc_tiling_on_sc=False
        ),
        scratch_types=[
            pltpu.HBM((NC, N_PAD, DH), jnp.float32),
            pltpu.VMEM_SHARED((N_PAD, DH), jnp.float32),
            pltpu.VMEM_SHARED((N_PAD,), jnp.float32),
            pltpu.VMEM((NCHUNK, K), jnp.int32),
            pltpu.VMEM((NCHUNK, K), jnp.int32),
            pltpu.VMEM((CH, DH), jnp.float32),
            pltpu.VMEM((K,), jnp.float32),
            pltpu.VMEM((RPT,), jnp.float32),
            pltpu.VMEM((RPT,), jnp.float32),
            pltpu.VMEM((RPT,), jnp.float32),
            pltpu.VMEM((RPT,), jnp.float32),
        ]
        + [pltpu.VMEM((K, DH), jnp.float32)] * NBUF
        + [pltpu.SemaphoreType.DMA] * (2 * NBUF),
    )
    return kern(x, row_h, col_h)


# speedup vs baseline: 1.2082x; 1.2082x over previous
import jax
import jax.numpy as jnp
from jax import lax
from jax.experimental import pallas as pl
from jax.experimental.pallas import tpu as pltpu
from jax.experimental.pallas import tpu_sc as plsc

N_NODES = 10000
N_PAD = 10240
D = 128
DH = 64
E = 320000
NC, NS, L = 2, 16, 16
EPT = E // NS
K = 80
NCHUNK = EPT // K
NBUF = 5
NGRP = NCHUNK // NBUF
RPT = N_PAD // NS
CH = 80
NRCH = RPT // CH
NSB = 4


def _rsqrt16(d):
    i = lax.bitcast_convert_type(d, jnp.int32)
    i = jnp.int32(0x5F3759DF) - lax.shift_right_logical(
        i, jnp.full((16,), 1, jnp.int32)
    )
    y = lax.bitcast_convert_type(i, jnp.float32)
    for _ in range(3):
        y = y * (1.5 - 0.5 * d * y * y)
    return jnp.where(d > 0.0, y, jnp.zeros((16,), jnp.float32))


def _bcast(scale_ref, i):
    return plsc.load_gather(scale_ref, [jnp.full((L,), i, jnp.int32)])


def _zero_buf(buf, rows):
    def body(r, carry):
        for f in range(DH // L):
            buf[r, pl.ds(f * L, L)] = jnp.zeros((L,), jnp.float32)
        return carry
    lax.fori_loop(0, rows, body, 0)


def _body(x_hbm, row_h, col_h, out_hbm,
          y_h, acc_s, deg_s,
          row_t, col_t, sbuf,
          ones_t, deg_t, dinv_t, dinv2_t, zvec,
          *bufsems):
    c = lax.axis_index("c")
    s = lax.axis_index("s")
    r0 = s * RPT
    f0 = c * DH
    bufs = bufsems[:NBUF]
    gsems = bufsems[NBUF:2 * NBUF]
    ssems = bufsems[2 * NBUF:3 * NBUF]

    pltpu.sync_copy(row_h.at[s], row_t)
    pltpu.sync_copy(col_h.at[s], col_t)

    for i in range(K // L):
        ones_t[pl.ds(i * L, L)] = jnp.ones((L,), jnp.float32)
    _zero_buf(sbuf, CH)

    def zv(i, carry):
        zvec[pl.ds(i * L, L)] = jnp.zeros((L,), jnp.float32)
        return carry
    lax.fori_loop(0, RPT // L, zv, 0)

    pltpu.sync_copy(zvec, deg_s.at[pl.ds(r0, RPT)])
    plsc.subcore_barrier()

    def deg_fire(j, carry):
        pltpu.async_copy(ones_t, deg_s.at[col_t.at[j]], gsems[0], add=True)
        return carry
    lax.fori_loop(0, NCHUNK, deg_fire, 0)

    def deg_drain(j, carry):
        pltpu.make_async_copy(ones_t, deg_s.at[col_t.at[0]], gsems[0]).wait()
        return carry
    lax.fori_loop(0, NCHUNK, deg_drain, 0)
    plsc.subcore_barrier()

    pltpu.sync_copy(deg_s.at[pl.ds(r0, RPT)], deg_t)
    for i in range(RPT // L):
        sl = pl.ds(i * L, L)
        r = _rsqrt16(deg_t[sl])
        dinv_t[sl] = r
        dinv2_t[sl] = r * r

    def node_pass(src_acc, last):
        scale_ref = dinv2_t if (src_acc and not last) else dinv_t

        def rd(ch, b):
            r = r0 + ch * CH
            if src_acc:
                pltpu.async_copy(acc_s.at[pl.ds(r, CH), :], bufs[b], gsems[b])
            else:
                pltpu.async_copy(
                    x_hbm.at[pl.ds(r, CH), pl.ds(f0, DH)], bufs[b], gsems[b]
                )

        def rd_wait(b):
            if src_acc:
                pltpu.make_async_copy(
                    acc_s.at[pl.ds(r0, CH), :], bufs[b], gsems[b]
                ).wait()
            else:
                pltpu.make_async_copy(
                    x_hbm.at[pl.ds(r0, CH), pl.ds(f0, DH)], bufs[b], gsems[b]
                ).wait()

        def wr(ch, b):
            r = r0 + ch * CH
            if last:
                pltpu.async_copy(
                    bufs[b], out_hbm.at[pl.ds(r, CH), pl.ds(f0, DH)], ssems[b]
                )
            else:
                pltpu.async_copy(bufs[b], y_h.at[c, pl.ds(r, CH), :], ssems[b])

        def wr_wait(b):
            if last:
                pltpu.make_async_copy(
                    bufs[b], out_hbm.at[pl.ds(r0, CH), pl.ds(f0, DH)], ssems[b]
                ).wait()
            else:
                pltpu.make_async_copy(
                    bufs[b], y_h.at[c, pl.ds(r0, CH), :], ssems[b]
                ).wait()

        for b in range(NSB):
            rd(b, b)

        def grp(g, carry):
            for b in range(NSB):
                ch = g * NSB + b
                rd_wait(b)
                if not last:
                    pltpu.async_copy(
                        sbuf, acc_s.at[pl.ds(r0 + ch * CH, CH), :], gsems[NSB]
                    )

                def rows(rr, c2):
                    v = _bcast(scale_ref, ch * CH + rr)
                    for f in range(DH // L):
                        sl = pl.ds(f * L, L)
                        bufs[b][rr, sl] = bufs[b][rr, sl] * v
                    return c2
                lax.fori_loop(0, CH, rows, 0)
                wr(ch, b)

                @pl.when(g < (NRCH // NSB) - 1)
                def _():
                    wr_wait(b)
                    rd(ch + NSB, b)
            return carry
        lax.fori_loop(0, NRCH // NSB, grp, 0)

        for b in range(NSB):
            wr_wait(b)
        if not last:
            def zdrain(j, carry):
                pltpu.make_async_copy(
                    sbuf, acc_s.at[pl.ds(r0, CH), :], gsems[NSB]
                ).wait()
                return carry
            lax.fori_loop(0, NRCH, zdrain, 0)

    node_pass(src_acc=False, last=False)
    plsc.subcore_barrier()

    def gfire(j, b):
        pltpu.async_copy(y_h.at[c].at[row_t.at[j]], bufs[b], gsems[b])

    def gwait(b):
        pltpu.make_async_copy(y_h.at[c].at[row_t.at[0]], bufs[b], gsems[b]).wait()

    def sfire(j, b):
        pltpu.async_copy(bufs[b], acc_s.at[col_t.at[j]], ssems[b], add=True)

    def swait(b):
        pltpu.make_async_copy(bufs[b], acc_s.at[col_t.at[0]], ssems[b]).wait()

    def edge_pass():
        for b in range(NBUF):
            gfire(jnp.int32(b), b)

        def grp(g, carry):
            j0 = g * NBUF
            for b in range(NBUF):
                gwait(b)
                sfire(j0 + b, b)
            for b in range(NBUF):
                swait(b)

                @pl.when(j0 + b + NBUF < NCHUNK)
                def _():
                    gfire(j0 + b + NBUF, b)
            return carry
        lax.fori_loop(0, NGRP, grp, 0)

    for t in range(3):
        edge_pass()
        plsc.subcore_barrier()
        node_pass(src_acc=True, last=(t == 2))
        if t < 2:
            plsc.subcore_barrier()


@jax.jit
def kernel(x, edge_index):
    ei = edge_index.astype(jnp.int32)
    row_h = ei[0].reshape(NS, NCHUNK, K)
    col_h = ei[1].reshape(NS, NCHUNK, K)
    xp = jnp.pad(x, ((0, N_PAD - N_NODES), (0, 0)))

    kern = pl.kernel(
        _body,
        out_type=jax.ShapeDtypeStruct((N_PAD, D), jnp.float32),
        mesh=plsc.VectorSubcoreMesh(core_axis_name="c", subcore_axis_name="s"),
        compiler_params=pltpu.CompilerParams(
            needs_layout_passes=False, use_tc_tiling_on_sc=False
        ),
        scratch_types=[
            pltpu.HBM((NC, N_PAD, DH), jnp.float32),
            pltpu.VMEM_SHARED((N_PAD, DH), jnp.float32),
            pltpu.VMEM_SHARED((N_PAD,), jnp.float32),
            pltpu.VMEM((NCHUNK, K), jnp.int32),
            pltpu.VMEM((NCHUNK, K), jnp.int32),
            pltpu.VMEM((CH, DH), jnp.float32),
            pltpu.VMEM((K,), jnp.float32),
            pltpu.VMEM((RPT,), jnp.float32),
            pltpu.VMEM((RPT,), jnp.float32),
            pltpu.VMEM((RPT,), jnp.float32),
            pltpu.VMEM((RPT,), jnp.float32),
        ]
        + [pltpu.VMEM((K, DH), jnp.float32)] * NBUF
        + [pltpu.SemaphoreType.DMA] * (2 * NBUF),
    )
    return kern(xp, row_h, col_h)[:N_NODES]
